# baseline (speedup 1.0000x reference)
import jax
import jax.numpy as jnp
from jax.experimental import pallas as pl
from jax.experimental.pallas import tpu as pltpu

DICT_LEN = 128
SEQ_LEN, BZ, D_MODEL, HIDDEN = 8192, 4, 768, 256
TILE_S = 1024
N_BLK = SEQ_LEN // TILE_S
NBUF = 8
N_SLAB = N_BLK * BZ


def _switcher_kernel(gid_ref, x_hbm, w1_ref, b1_ref, w2_ref, b2_ref, o_hbm,
                     xbuf, obuf, in_sems, out_sems):
    i = pl.program_id(0)

    def copy_in(k, slot):
        s, b = k // BZ, k % BZ
        return pltpu.make_async_copy(
            x_hbm.at[pl.ds(s * TILE_S, TILE_S), b, :],
            xbuf.at[slot],
            in_sems.at[slot],
        )

    def copy_out(k, slot):
        s, b = k // BZ, k % BZ
        return pltpu.make_async_copy(
            obuf.at[slot],
            o_hbm.at[pl.ds(s * TILE_S, TILE_S), b, :],
            out_sems.at[slot],
        )

    @pl.when(i == 0)
    def _():
        for k in range(NBUF - 1):
            copy_in(k, k).start()

    @pl.when(i + NBUF - 1 < N_SLAB)
    def _():
        copy_in(i + NBUF - 1, (i + NBUF - 1) % NBUF).start()

    copy_in(i, i % NBUF).wait()

    @pl.when(i >= NBUF)
    def _():
        copy_out(i - NBUF, i % NBUF).wait()

    g = gid_ref[i % BZ]
    x = xbuf[i % NBUF]
    h = jnp.dot(x, w1_ref[g], preferred_element_type=jnp.float32)
    h = h + b1_ref[g]
    h = 0.5 * h * (1.0 + jax.lax.erf(h * 0.7071067811865476))
    out = jnp.dot(h, w2_ref[g], preferred_element_type=jnp.float32)
    obuf[i % NBUF] = out + b2_ref[g]

    copy_out(i, i % NBUF).start()

    @pl.when(i == N_SLAB - 1)
    def _():
        for k in range(N_SLAB - NBUF, N_SLAB):
            copy_out(k, k % NBUF).wait()


def kernel(x, lang_ids, w1_0, b1_0, w2_0, b2_0, w1_1, b1_1, w2_1, b2_1):
    gid = (DICT_LEN - 1 - lang_ids.astype(jnp.int32) <= 3).astype(jnp.int32)
    w1 = jnp.stack([w1_0.T, w1_1.T])
    b1 = jnp.stack([b1_0, b1_1])[:, None, :]
    w2 = jnp.stack([w2_0.T, w2_1.T])
    b2 = jnp.stack([b2_0, b2_1])[:, None, :]
    return pl.pallas_call(
        _switcher_kernel,
        grid_spec=pltpu.PrefetchScalarGridSpec(
            num_scalar_prefetch=1,
            grid=(N_SLAB,),
            in_specs=[
                pl.BlockSpec(memory_space=pl.ANY),
                pl.BlockSpec((2, D_MODEL, HIDDEN), lambda i, g: (0, 0, 0)),
                pl.BlockSpec((2, 1, HIDDEN), lambda i, g: (0, 0, 0)),
                pl.BlockSpec((2, HIDDEN, D_MODEL), lambda i, g: (0, 0, 0)),
                pl.BlockSpec((2, 1, D_MODEL), lambda i, g: (0, 0, 0)),
            ],
            out_specs=pl.BlockSpec(memory_space=pl.ANY),
            scratch_shapes=[
                pltpu.VMEM((NBUF, TILE_S, D_MODEL), jnp.float32),
                pltpu.VMEM((NBUF, TILE_S, D_MODEL), jnp.float32),
                pltpu.SemaphoreType.DMA((NBUF,)),
                pltpu.SemaphoreType.DMA((NBUF,)),
            ],
        ),
        out_shape=jax.ShapeDtypeStruct(x.shape, x.dtype),
        compiler_params=pltpu.CompilerParams(
            dimension_semantics=("arbitrary",),
        ),
    )(gid, x, w1, b1, w2, b2)


# speedup vs baseline: 1.0235x; 1.0235x over previous
import jax
import jax.numpy as jnp
from jax.experimental import pallas as pl
from jax.experimental.pallas import tpu as pltpu

DICT_LEN = 128
SEQ_LEN, BZ, D_MODEL, HIDDEN = 8192, 4, 768, 256
TILE_S = 2048
N_BLK = SEQ_LEN // TILE_S
NBUF = 4
N_SLAB = N_BLK * BZ


def _switcher_kernel(gid_ref, x_hbm, w1_ref, b1_ref, w2_ref, b2_ref, o_hbm,
                     xbuf, obuf, in_sems, out_sems):
    i = pl.program_id(0)

    def copy_in(k, slot):
        s, b = k // BZ, k % BZ
        return pltpu.make_async_copy(
            x_hbm.at[pl.ds(s * TILE_S, TILE_S), b, :],
            xbuf.at[slot],
            in_sems.at[slot],
        )

    def copy_out(k, slot):
        s, b = k // BZ, k % BZ
        return pltpu.make_async_copy(
            obuf.at[slot],
            o_hbm.at[pl.ds(s * TILE_S, TILE_S), b, :],
            out_sems.at[slot],
        )

    @pl.when(i == 0)
    def _():
        for k in range(NBUF - 1):
            copy_in(k, k).start()

    @pl.when(i + NBUF - 1 < N_SLAB)
    def _():
        copy_in(i + NBUF - 1, (i + NBUF - 1) % NBUF).start()

    copy_in(i, i % NBUF).wait()

    @pl.when(i >= NBUF)
    def _():
        copy_out(i - NBUF, i % NBUF).wait()

    g = gid_ref[i % BZ]
    x = xbuf[i % NBUF]
    h = jnp.dot(x, w1_ref[g], preferred_element_type=jnp.float32)
    h = h + b1_ref[g]
    h = 0.5 * h * (1.0 + jax.lax.erf(h * 0.7071067811865476))
    out = jnp.dot(h, w2_ref[g], preferred_element_type=jnp.float32)
    obuf[i % NBUF] = out + b2_ref[g]

    copy_out(i, i % NBUF).start()

    @pl.when(i == N_SLAB - 1)
    def _():
        for k in range(N_SLAB - NBUF, N_SLAB):
            copy_out(k, k % NBUF).wait()


def kernel(x, lang_ids, w1_0, b1_0, w2_0, b2_0, w1_1, b1_1, w2_1, b2_1):
    gid = (DICT_LEN - 1 - lang_ids.astype(jnp.int32) <= 3).astype(jnp.int32)
    w1 = jnp.stack([w1_0.T, w1_1.T])
    b1 = jnp.stack([b1_0, b1_1])[:, None, :]
    w2 = jnp.stack([w2_0.T, w2_1.T])
    b2 = jnp.stack([b2_0, b2_1])[:, None, :]
    return pl.pallas_call(
        _switcher_kernel,
        grid_spec=pltpu.PrefetchScalarGridSpec(
            num_scalar_prefetch=1,
            grid=(N_SLAB,),
            in_specs=[
                pl.BlockSpec(memory_space=pl.ANY),
                pl.BlockSpec((2, D_MODEL, HIDDEN), lambda i, g: (0, 0, 0)),
                pl.BlockSpec((2, 1, HIDDEN), lambda i, g: (0, 0, 0)),
                pl.BlockSpec((2, HIDDEN, D_MODEL), lambda i, g: (0, 0, 0)),
                pl.BlockSpec((2, 1, D_MODEL), lambda i, g: (0, 0, 0)),
            ],
            out_specs=pl.BlockSpec(memory_space=pl.ANY),
            scratch_shapes=[
                pltpu.VMEM((NBUF, TILE_S, D_MODEL), jnp.float32),
                pltpu.VMEM((NBUF, TILE_S, D_MODEL), jnp.float32),
                pltpu.SemaphoreType.DMA((NBUF,)),
                pltpu.SemaphoreType.DMA((NBUF,)),
            ],
        ),
        out_shape=jax.ShapeDtypeStruct(x.shape, x.dtype),
        compiler_params=pltpu.CompilerParams(
            dimension_semantics=("arbitrary",),
        ),
    )(gid, x, w1, b1, w2, b2)
